# baseline (speedup 1.0000x reference)
import functools

import jax
import jax.numpy as jnp
from jax import lax
from jax.experimental import pallas as pl
from jax.experimental.pallas import tpu as pltpu
from jax.experimental.pallas import tpu_sc as plsc

N = 10000
D = 128
E = 320000
NC = 2
NS = 16
NW = NC * NS
C = 128
J = 80
E_PAD = NW * J * C
N_SUB = 640
N_PAD = NS * N_SUB
CW = 128


def _agg_body(x_hbm, src_hbm, dst_hbm, z_hbm,
              sums_hbm,
              acc_sh, sidx, dring, rows0, rows1, sg0, sg1, si0, si1):
    cid = lax.axis_index("c")
    sid = lax.axis_index("s")
    wid = sid * NC + cid
    r0 = sid * N_SUB
    pltpu.sync_copy(z_hbm, acc_sh.at[pl.ds(r0, N_SUB)])
    pltpu.sync_copy(src_hbm.at[wid], sidx)
    plsc.subcore_barrier()

    rows = (rows0, rows1)
    sg = (sg0, sg1)
    si = (si0, si1)

    for b in range(2):
        pltpu.async_copy(dst_hbm.at[wid, b], dring.at[b], si[b])
        pltpu.async_copy(x_hbm.at[sidx.at[b]], rows[b], sg[b])

    def body(k, carry):
        for b in range(2):
            jj = 2 * k + b
            pltpu.make_async_copy(dst_hbm.at[wid, jj], dring.at[b], si[b]).wait()
            pltpu.make_async_copy(x_hbm.at[sidx.at[jj]], rows[b], sg[b]).wait()
            pltpu.sync_copy(rows[b], acc_sh.at[dring.at[b]], add=True)

            @pl.when(jj + 2 < J)
            def _():
                pltpu.async_copy(dst_hbm.at[wid, jj + 2], dring.at[b], si[b])
                pltpu.async_copy(x_hbm.at[sidx.at[jj + 2]], rows[b], sg[b])
        return carry

    lax.fori_loop(0, J // 2, body, 0)
    plsc.subcore_barrier()
    pltpu.sync_copy(acc_sh.at[pl.ds(r0, N_SUB)], sums_hbm.at[cid, pl.ds(r0, N_SUB)])


def _count_body(dst_hbm, z_hbm, ones_hbm,
                cnt_hbm,
                cnt_sh, didx, ones_v, sem):
    cid = lax.axis_index("c")
    sid = lax.axis_index("s")
    wid = sid * NC + cid
    r0 = sid * N_SUB
    pltpu.sync_copy(z_hbm, cnt_sh.at[pl.ds(r0, N_SUB)])
    pltpu.sync_copy(dst_hbm.at[wid], didx)
    pltpu.sync_copy(ones_hbm, ones_v)
    plsc.subcore_barrier()

    def fire(j, carry):
        pltpu.async_copy(ones_v, cnt_sh.at[didx.at[j]], sem, add=True)
        return carry

    lax.fori_loop(0, J, fire, 0)

    def drain(j, carry):
        pltpu.make_async_copy(ones_v, cnt_sh.at[didx.at[0]], sem).wait()
        return carry

    lax.fori_loop(0, J, drain, 0)
    plsc.subcore_barrier()
    pltpu.sync_copy(cnt_sh.at[pl.ds(r0, N_SUB)], cnt_hbm.at[cid, pl.ds(r0, N_SUB)])


_MESH = plsc.VectorSubcoreMesh(core_axis_name="c", subcore_axis_name="s")

_agg = pl.kernel(
    _agg_body,
    mesh=_MESH,
    out_type=[jax.ShapeDtypeStruct((NC, N_PAD, D), jnp.float32)],
    scratch_types=[
        pltpu.VMEM_SHARED((N_PAD, D), jnp.float32),
        pltpu.VMEM((J, C), jnp.int32),
        pltpu.VMEM((2, C), jnp.int32),
        pltpu.VMEM((C, D), jnp.float32),
        pltpu.VMEM((C, D), jnp.float32),
        pltpu.SemaphoreType.DMA,
        pltpu.SemaphoreType.DMA,
        pltpu.SemaphoreType.DMA,
        pltpu.SemaphoreType.DMA,
    ],
)

_count = pl.kernel(
    _count_body,
    mesh=_MESH,
    out_type=[jax.ShapeDtypeStruct((NC, N_PAD, CW), jnp.float32)],
    scratch_types=[
        pltpu.VMEM_SHARED((N_PAD, CW), jnp.float32),
        pltpu.VMEM((J, C), jnp.int32),
        pltpu.VMEM((C, CW), jnp.float32),
        pltpu.SemaphoreType.DMA,
    ],
)


def _dense_kernel(relu, s0_ref, s1_ref, c0_ref, c1_ref, x_ref, wl_ref, b_ref,
                  wr_ref, o_ref):
    cnt = c0_ref[:, 0:1] + c1_ref[:, 0:1]
    mean = (s0_ref[...] + s1_ref[...]) / jnp.maximum(cnt, 1.0)
    r = (jnp.dot(mean, wl_ref[...], preferred_element_type=jnp.float32)
         + b_ref[...]
         + jnp.dot(x_ref[...], wr_ref[...], preferred_element_type=jnp.float32))
    if relu:
        r = jnp.maximum(r, 0.0)
    o_ref[...] = r


_BLK = 2000


def _dense(s0, s1, c0, c1, x, wl_t, b, wr_t, relu):
    grid = (N // _BLK,)
    row_spec = pl.BlockSpec((_BLK, D), lambda i: (i, 0))
    cnt_spec = pl.BlockSpec((_BLK, CW), lambda i: (i, 0))
    w_spec = pl.BlockSpec((D, D), lambda i: (0, 0))
    b_spec = pl.BlockSpec((1, D), lambda i: (0, 0))
    return pl.pallas_call(
        functools.partial(_dense_kernel, relu),
        grid=grid,
        in_specs=[row_spec, row_spec, cnt_spec, cnt_spec, row_spec, w_spec,
                  b_spec, w_spec],
        out_specs=row_spec,
        out_shape=jax.ShapeDtypeStruct((N, D), jnp.float32),
    )(s0, s1, c0, c1, x, wl_t, b, wr_t)


def kernel(x, edge_index, W1_l, b1_l, W1_r, W2_l, b2_l, W2_r):
    src = edge_index[0]
    dst = edge_index[1]
    npad = E_PAD - E
    pad_src = (jnp.arange(npad, dtype=jnp.int32) * 37) % N
    pad_dst = N + (jnp.arange(npad, dtype=jnp.int32) % (N_PAD - N))
    src_p = jnp.concatenate([src, pad_src]).reshape(NW, J, C)
    dst_p = jnp.concatenate([dst, pad_dst]).reshape(NW, J, C)

    zeros = jnp.zeros((N_SUB, D), jnp.float32)
    ones = jnp.ones((C, CW), jnp.float32)

    (cnts,) = _count(dst_p, zeros, ones)
    (sums1,) = _agg(x, src_p, dst_p, zeros)
    h = _dense(sums1[0], sums1[1], cnts[0], cnts[1], x,
               W1_l.T, b1_l.reshape(1, D), W1_r.T, relu=True)
    (sums2,) = _agg(h, src_p, dst_p, zeros)
    out = _dense(sums2[0], sums2[1], cnts[0], cnts[1], h,
                 W2_l.T, b2_l.reshape(1, D), W2_r.T, relu=False)
    return out


# speedup vs baseline: 1.0547x; 1.0547x over previous
import functools

import jax
import jax.numpy as jnp
from jax import lax
from jax.experimental import pallas as pl
from jax.experimental.pallas import tpu as pltpu
from jax.experimental.pallas import tpu_sc as plsc

N = 10000
D = 128
E = 320000
NC = 2
NS = 16
NW = NC * NS
C = 128
J = 80
E_PAD = NW * J * C
N_SUB = 640
N_PAD = NS * N_SUB
CW = 128


def _agg_body(x_hbm, src_hbm, dst_hbm, z_hbm,
              sums_hbm,
              acc_sh, sidx, dring, rows0, rows1, sg0, sg1, si0, si1):
    cid = lax.axis_index("c")
    sid = lax.axis_index("s")
    wid = sid * NC + cid
    r0 = sid * N_SUB
    pltpu.sync_copy(z_hbm, acc_sh.at[pl.ds(r0, N_SUB)])
    pltpu.sync_copy(src_hbm.at[wid], sidx)
    plsc.subcore_barrier()

    rows = (rows0, rows1)
    sg = (sg0, sg1)
    si = (si0, si1)

    for b in range(2):
        pltpu.async_copy(dst_hbm.at[wid, b], dring.at[b], si[b])
        pltpu.async_copy(x_hbm.at[sidx.at[b]], rows[b], sg[b])

    def body(k, carry):
        for b in range(2):
            jj = 2 * k + b
            pltpu.make_async_copy(dst_hbm.at[wid, jj], dring.at[b], si[b]).wait()
            pltpu.make_async_copy(x_hbm.at[sidx.at[jj]], rows[b], sg[b]).wait()
            pltpu.sync_copy(rows[b], acc_sh.at[dring.at[b]], add=True)

            @pl.when(jj + 2 < J)
            def _():
                pltpu.async_copy(dst_hbm.at[wid, jj + 2], dring.at[b], si[b])
                pltpu.async_copy(x_hbm.at[sidx.at[jj + 2]], rows[b], sg[b])
        return carry

    lax.fori_loop(0, J // 2, body, 0)
    plsc.subcore_barrier()
    pltpu.sync_copy(acc_sh.at[pl.ds(r0, N_SUB)], sums_hbm.at[cid, pl.ds(r0, N_SUB)])


def _count_body(dst_hbm, z_hbm, ones_hbm,
                cnt_hbm,
                cnt_sh, didx, ones_v, sem):
    cid = lax.axis_index("c")
    sid = lax.axis_index("s")
    wid = sid * NC + cid
    r0 = sid * N_SUB
    pltpu.sync_copy(z_hbm, cnt_sh.at[pl.ds(r0, N_SUB)])
    pltpu.sync_copy(dst_hbm.at[wid], didx)
    pltpu.sync_copy(ones_hbm, ones_v)
    plsc.subcore_barrier()

    def fire(j, carry):
        pltpu.async_copy(ones_v, cnt_sh.at[didx.at[j]], sem, add=True)
        return carry

    lax.fori_loop(0, J, fire, 0)

    def drain(j, carry):
        pltpu.make_async_copy(ones_v, cnt_sh.at[didx.at[0]], sem).wait()
        return carry

    lax.fori_loop(0, J, drain, 0)
    plsc.subcore_barrier()
    pltpu.sync_copy(cnt_sh.at[pl.ds(r0, N_SUB)], cnt_hbm.at[cid, pl.ds(r0, N_SUB)])


_MESH = plsc.VectorSubcoreMesh(core_axis_name="c", subcore_axis_name="s")

_agg = pl.kernel(
    _agg_body,
    mesh=_MESH,
    out_type=[jax.ShapeDtypeStruct((NC, N_PAD, D), jnp.float32)],
    scratch_types=[
        pltpu.VMEM_SHARED((N_PAD, D), jnp.float32),
        pltpu.VMEM((J, C), jnp.int32),
        pltpu.VMEM((2, C), jnp.int32),
        pltpu.VMEM((C, D), jnp.float32),
        pltpu.VMEM((C, D), jnp.float32),
        pltpu.SemaphoreType.DMA,
        pltpu.SemaphoreType.DMA,
        pltpu.SemaphoreType.DMA,
        pltpu.SemaphoreType.DMA,
    ],
)

_count = pl.kernel(
    _count_body,
    mesh=_MESH,
    out_type=[jax.ShapeDtypeStruct((NC, N_PAD, CW), jnp.float32)],
    scratch_types=[
        pltpu.VMEM_SHARED((N_PAD, CW), jnp.float32),
        pltpu.VMEM((J, C), jnp.int32),
        pltpu.VMEM((C, CW), jnp.float32),
        pltpu.SemaphoreType.DMA,
    ],
)


def _dense_kernel(relu, s_ref, c_ref, x_ref, wl_ref, b_ref, wr_ref, o_ref):
    cnt = c_ref[0, :, 0:1] + c_ref[1, :, 0:1]
    mean = (s_ref[0] + s_ref[1]) / jnp.maximum(cnt, 1.0)
    r = (jnp.dot(mean, wl_ref[...], preferred_element_type=jnp.float32)
         + b_ref[...]
         + jnp.dot(x_ref[...], wr_ref[...], preferred_element_type=jnp.float32))
    if relu:
        r = jnp.maximum(r, 0.0)
    o_ref[...] = r


_BLK = 2000


def _dense(sums, cnts, x, wl_t, b, wr_t, relu):
    grid = (N // _BLK,)
    row_spec = pl.BlockSpec((_BLK, D), lambda i: (i, 0))
    sum_spec = pl.BlockSpec((NC, _BLK, D), lambda i: (0, i, 0))
    cnt_spec = pl.BlockSpec((NC, _BLK, CW), lambda i: (0, i, 0))
    w_spec = pl.BlockSpec((D, D), lambda i: (0, 0))
    b_spec = pl.BlockSpec((1, D), lambda i: (0, 0))
    return pl.pallas_call(
        functools.partial(_dense_kernel, relu),
        grid=grid,
        in_specs=[sum_spec, cnt_spec, row_spec, w_spec, b_spec, w_spec],
        out_specs=row_spec,
        out_shape=jax.ShapeDtypeStruct((N, D), jnp.float32),
    )(sums, cnts, x, wl_t, b, wr_t)


def kernel(x, edge_index, W1_l, b1_l, W1_r, W2_l, b2_l, W2_r):
    src = edge_index[0]
    dst = edge_index[1]
    npad = E_PAD - E
    pad_src = (jnp.arange(npad, dtype=jnp.int32) * 37) % N
    pad_dst = N + (jnp.arange(npad, dtype=jnp.int32) % (N_PAD - N))
    src_p = jnp.concatenate([src, pad_src]).reshape(NW, J, C)
    dst_p = jnp.concatenate([dst, pad_dst]).reshape(NW, J, C)

    zeros = jnp.zeros((N_SUB, D), jnp.float32)
    ones = jnp.ones((C, CW), jnp.float32)

    (cnts,) = _count(dst_p, zeros, ones)
    (sums1,) = _agg(x, src_p, dst_p, zeros)
    h = _dense(sums1, cnts, x, W1_l.T, b1_l.reshape(1, D), W1_r.T, relu=True)
    (sums2,) = _agg(h, src_p, dst_p, zeros)
    out = _dense(sums2, cnts, h, W2_l.T, b2_l.reshape(1, D), W2_r.T, relu=False)
    return out


# speedup vs baseline: 1.2458x; 1.1812x over previous
import functools

import jax
import jax.numpy as jnp
from jax import lax
from jax.experimental import pallas as pl
from jax.experimental.pallas import tpu as pltpu
from jax.experimental.pallas import tpu_sc as plsc

N = 10000
D = 128
E = 320000
NC = 2
NS = 16
NW = NC * NS
C = 128
J = 80
E_PAD = NW * J * C
N_SUB = 640
N_PAD = NS * N_SUB
CW = 16


def _agg_body(x_hbm, src_hbm, dst_hbm, z_hbm,
              sums_hbm,
              acc_sh, sidx, dring, rows0, rows1, sg0, sg1, si0, si1):
    cid = lax.axis_index("c")
    sid = lax.axis_index("s")
    wid = sid * NC + cid
    r0 = sid * N_SUB
    pltpu.sync_copy(z_hbm, acc_sh.at[pl.ds(r0, N_SUB)])
    pltpu.sync_copy(src_hbm.at[wid], sidx)
    plsc.subcore_barrier()

    rows = (rows0, rows1)
    sg = (sg0, sg1)
    si = (si0, si1)

    for b in range(2):
        pltpu.async_copy(dst_hbm.at[wid, b], dring.at[b], si[b])
        pltpu.async_copy(x_hbm.at[sidx.at[b]], rows[b], sg[b])

    def body(k, carry):
        for b in range(2):
            jj = 2 * k + b
            pltpu.make_async_copy(dst_hbm.at[wid, jj], dring.at[b], si[b]).wait()
            pltpu.make_async_copy(x_hbm.at[sidx.at[jj]], rows[b], sg[b]).wait()
            pltpu.sync_copy(rows[b], acc_sh.at[dring.at[b]], add=True)

            @pl.when(jj + 2 < J)
            def _():
                pltpu.async_copy(dst_hbm.at[wid, jj + 2], dring.at[b], si[b])
                pltpu.async_copy(x_hbm.at[sidx.at[jj + 2]], rows[b], sg[b])
        return carry

    lax.fori_loop(0, J // 2, body, 0)
    plsc.subcore_barrier()
    pltpu.sync_copy(acc_sh.at[pl.ds(r0, N_SUB)], sums_hbm.at[cid, pl.ds(r0, N_SUB)])


def _count_body(dst_hbm, z_hbm, ones_hbm,
                cnt_hbm,
                cnt_sh, didx, ones_v, sem):
    cid = lax.axis_index("c")
    sid = lax.axis_index("s")
    wid = sid * NC + cid
    r0 = sid * N_SUB
    pltpu.sync_copy(z_hbm, cnt_sh.at[pl.ds(r0, N_SUB)])
    pltpu.sync_copy(dst_hbm.at[wid], didx)
    pltpu.sync_copy(ones_hbm, ones_v)
    plsc.subcore_barrier()

    def fire(j, carry):
        pltpu.async_copy(ones_v, cnt_sh.at[didx.at[j]], sem, add=True)
        return carry

    lax.fori_loop(0, J, fire, 0)

    def drain(j, carry):
        pltpu.make_async_copy(ones_v, cnt_sh.at[didx.at[0]], sem).wait()
        return carry

    lax.fori_loop(0, J, drain, 0)
    plsc.subcore_barrier()
    pltpu.sync_copy(cnt_sh.at[pl.ds(r0, N_SUB)], cnt_hbm.at[cid, pl.ds(r0, N_SUB)])


_MESH = plsc.VectorSubcoreMesh(core_axis_name="c", subcore_axis_name="s")

_agg = pl.kernel(
    _agg_body,
    mesh=_MESH,
    out_type=[jax.ShapeDtypeStruct((NC, N_PAD, D), jnp.float32)],
    scratch_types=[
        pltpu.VMEM_SHARED((N_PAD, D), jnp.float32),
        pltpu.VMEM((J, C), jnp.int32),
        pltpu.VMEM((2, C), jnp.int32),
        pltpu.VMEM((C, D), jnp.float32),
        pltpu.VMEM((C, D), jnp.float32),
        pltpu.SemaphoreType.DMA,
        pltpu.SemaphoreType.DMA,
        pltpu.SemaphoreType.DMA,
        pltpu.SemaphoreType.DMA,
    ],
)

_count = pl.kernel(
    _count_body,
    mesh=_MESH,
    out_type=[jax.ShapeDtypeStruct((NC, N_PAD, CW), jnp.float32)],
    scratch_types=[
        pltpu.VMEM_SHARED((N_PAD, CW), jnp.float32),
        pltpu.VMEM((J, C), jnp.int32),
        pltpu.VMEM((C, CW), jnp.float32),
        pltpu.SemaphoreType.DMA,
    ],
    compiler_params=pltpu.CompilerParams(use_tc_tiling_on_sc=False),
)


def _dense_kernel(relu, s_ref, c_ref, x_ref, wl_ref, b_ref, wr_ref, o_ref):
    cnt = c_ref[0, :, 0:1] + c_ref[1, :, 0:1]
    mean = (s_ref[0] + s_ref[1]) / jnp.maximum(cnt, 1.0)
    r = (jnp.dot(mean, wl_ref[...], preferred_element_type=jnp.float32)
         + b_ref[...]
         + jnp.dot(x_ref[...], wr_ref[...], preferred_element_type=jnp.float32))
    if relu:
        r = jnp.maximum(r, 0.0)
    o_ref[...] = r


_BLK = 2000


def _dense(sums, cnts, x, wl_t, b, wr_t, relu):
    grid = (N // _BLK,)
    row_spec = pl.BlockSpec((_BLK, D), lambda i: (i, 0))
    sum_spec = pl.BlockSpec((NC, _BLK, D), lambda i: (0, i, 0))
    cnt_spec = pl.BlockSpec((NC, _BLK, CW), lambda i: (0, i, 0))
    w_spec = pl.BlockSpec((D, D), lambda i: (0, 0))
    b_spec = pl.BlockSpec((1, D), lambda i: (0, 0))
    return pl.pallas_call(
        functools.partial(_dense_kernel, relu),
        grid=grid,
        in_specs=[sum_spec, cnt_spec, row_spec, w_spec, b_spec, w_spec],
        out_specs=row_spec,
        out_shape=jax.ShapeDtypeStruct((N, D), jnp.float32),
    )(sums, cnts, x, wl_t, b, wr_t)


def kernel(x, edge_index, W1_l, b1_l, W1_r, W2_l, b2_l, W2_r):
    src = edge_index[0]
    dst = edge_index[1]
    npad = E_PAD - E
    pad_src = (jnp.arange(npad, dtype=jnp.int32) * 37) % N
    pad_dst = N + (jnp.arange(npad, dtype=jnp.int32) % (N_PAD - N))
    src_p = jnp.concatenate([src, pad_src]).reshape(NW, J, C)
    dst_p = jnp.concatenate([dst, pad_dst]).reshape(NW, J, C)

    zeros = jnp.zeros((N_SUB, D), jnp.float32)
    zeros_c = jnp.zeros((N_SUB, CW), jnp.float32)
    ones = jnp.ones((C, CW), jnp.float32)

    (cnts,) = _count(dst_p, zeros_c, ones)
    (sums1,) = _agg(x, src_p, dst_p, zeros)
    h = _dense(sums1, cnts, x, W1_l.T, b1_l.reshape(1, D), W1_r.T, relu=True)
    (sums2,) = _agg(h, src_p, dst_p, zeros)
    out = _dense(sums2, cnts, h, W2_l.T, b2_l.reshape(1, D), W2_r.T, relu=False)
    return out


# speedup vs baseline: 1.2467x; 1.0007x over previous
import functools

import jax
import jax.numpy as jnp
from jax import lax
from jax.experimental import pallas as pl
from jax.experimental.pallas import tpu as pltpu
from jax.experimental.pallas import tpu_sc as plsc

N = 10000
D = 128
E = 320000
NC = 2
NS = 16
NW = NC * NS
C = 128
J = 80
E_PAD = NW * J * C
N_SUB = 640
N_PAD = NS * N_SUB
CW = 16


def _agg_body(x_hbm, src_hbm, dst_hbm, z_hbm,
              sums_hbm,
              acc_sh, sidx, dring, rows0, rows1, sg0, sg1, si0, si1):
    cid = lax.axis_index("c")
    sid = lax.axis_index("s")
    wid = sid * NC + cid
    r0 = sid * N_SUB
    e0 = wid * (J * C)
    pltpu.sync_copy(z_hbm, acc_sh.at[pl.ds(r0, N_SUB)])
    pltpu.sync_copy(src_hbm.at[pl.ds(e0, J * C)], sidx)
    plsc.subcore_barrier()

    rows = (rows0, rows1)
    sg = (sg0, sg1)
    si = (si0, si1)

    for b in range(2):
        pltpu.async_copy(dst_hbm.at[pl.ds(e0 + b * C, C)], dring.at[b], si[b])
        pltpu.async_copy(x_hbm.at[sidx.at[pl.ds(b * C, C)]], rows[b], sg[b])

    def body(k, carry):
        for b in range(2):
            jj = 2 * k + b
            pltpu.make_async_copy(dst_hbm.at[pl.ds(e0 + jj * C, C)],
                                  dring.at[b], si[b]).wait()
            pltpu.make_async_copy(x_hbm.at[sidx.at[pl.ds(jj * C, C)]],
                                  rows[b], sg[b]).wait()
            pltpu.sync_copy(rows[b], acc_sh.at[dring.at[b]], add=True)

            @pl.when(jj + 2 < J)
            def _():
                pltpu.async_copy(dst_hbm.at[pl.ds(e0 + (jj + 2) * C, C)],
                                 dring.at[b], si[b])
                pltpu.async_copy(x_hbm.at[sidx.at[pl.ds((jj + 2) * C, C)]],
                                 rows[b], sg[b])
        return carry

    lax.fori_loop(0, J // 2, body, 0)
    plsc.subcore_barrier()
    pltpu.sync_copy(acc_sh.at[pl.ds(r0, N_SUB)], sums_hbm.at[cid, pl.ds(r0, N_SUB)])


def _count_body(dst_hbm, z_hbm, ones_hbm,
                cnt_hbm,
                cnt_sh, didx, ones_v, sem):
    cid = lax.axis_index("c")
    sid = lax.axis_index("s")
    wid = sid * NC + cid
    r0 = sid * N_SUB
    pltpu.sync_copy(z_hbm, cnt_sh.at[pl.ds(r0, N_SUB)])
    pltpu.sync_copy(dst_hbm.at[pl.ds(wid * (J * C), J * C)], didx)
    pltpu.sync_copy(ones_hbm, ones_v)
    plsc.subcore_barrier()

    def fire(j, carry):
        pltpu.async_copy(ones_v, cnt_sh.at[didx.at[pl.ds(j * C, C)]], sem,
                         add=True)
        return carry

    lax.fori_loop(0, J, fire, 0)

    def drain(j, carry):
        pltpu.make_async_copy(ones_v, cnt_sh.at[didx.at[pl.ds(0, C)]],
                              sem).wait()
        return carry

    lax.fori_loop(0, J, drain, 0)
    plsc.subcore_barrier()
    pltpu.sync_copy(cnt_sh.at[pl.ds(r0, N_SUB)], cnt_hbm.at[cid, pl.ds(r0, N_SUB)])


_MESH = plsc.VectorSubcoreMesh(core_axis_name="c", subcore_axis_name="s")

_agg = pl.kernel(
    _agg_body,
    mesh=_MESH,
    out_type=[jax.ShapeDtypeStruct((NC, N_PAD, D), jnp.float32)],
    scratch_types=[
        pltpu.VMEM_SHARED((N_PAD, D), jnp.float32),
        pltpu.VMEM((J * C,), jnp.int32),
        pltpu.VMEM((2, C), jnp.int32),
        pltpu.VMEM((C, D), jnp.float32),
        pltpu.VMEM((C, D), jnp.float32),
        pltpu.SemaphoreType.DMA,
        pltpu.SemaphoreType.DMA,
        pltpu.SemaphoreType.DMA,
        pltpu.SemaphoreType.DMA,
    ],
)

_count = pl.kernel(
    _count_body,
    mesh=_MESH,
    out_type=[jax.ShapeDtypeStruct((NC, N_PAD, CW), jnp.float32)],
    scratch_types=[
        pltpu.VMEM_SHARED((N_PAD, CW), jnp.float32),
        pltpu.VMEM((J * C,), jnp.int32),
        pltpu.VMEM((C, CW), jnp.float32),
        pltpu.SemaphoreType.DMA,
    ],
    compiler_params=pltpu.CompilerParams(use_tc_tiling_on_sc=False),
)


def _dense_kernel(relu, s_ref, c_ref, x_ref, wl_ref, b_ref, wr_ref, o_ref):
    cnt = c_ref[0, :, 0:1] + c_ref[1, :, 0:1]
    mean = (s_ref[0] + s_ref[1]) / jnp.maximum(cnt, 1.0)
    r = (jnp.dot(mean, wl_ref[...], preferred_element_type=jnp.float32)
         + b_ref[...]
         + jnp.dot(x_ref[...], wr_ref[...], preferred_element_type=jnp.float32))
    if relu:
        r = jnp.maximum(r, 0.0)
    o_ref[...] = r


_BLK = 2000


def _dense(sums, cnts, x, wl_t, b, wr_t, relu):
    grid = (N // _BLK,)
    row_spec = pl.BlockSpec((_BLK, D), lambda i: (i, 0))
    sum_spec = pl.BlockSpec((NC, _BLK, D), lambda i: (0, i, 0))
    cnt_spec = pl.BlockSpec((NC, _BLK, CW), lambda i: (0, i, 0))
    w_spec = pl.BlockSpec((D, D), lambda i: (0, 0))
    b_spec = pl.BlockSpec((1, D), lambda i: (0, 0))
    return pl.pallas_call(
        functools.partial(_dense_kernel, relu),
        grid=grid,
        in_specs=[sum_spec, cnt_spec, row_spec, w_spec, b_spec, w_spec],
        out_specs=row_spec,
        out_shape=jax.ShapeDtypeStruct((N, D), jnp.float32),
    )(sums, cnts, x, wl_t, b, wr_t)


def kernel(x, edge_index, W1_l, b1_l, W1_r, W2_l, b2_l, W2_r):
    src = edge_index[0]
    dst = edge_index[1]
    npad = E_PAD - E
    pad_src = jnp.arange(npad, dtype=jnp.int32) & 8191
    pad_dst = N + (jnp.arange(npad, dtype=jnp.int32) & 127)
    src_p = jnp.concatenate([src, pad_src])
    dst_p = jnp.concatenate([dst, pad_dst])

    zeros = jnp.zeros((N_SUB, D), jnp.float32)
    zeros_c = jnp.zeros((N_SUB, CW), jnp.float32)
    ones = jnp.ones((C, CW), jnp.float32)

    (cnts,) = _count(dst_p, zeros_c, ones)
    (sums1,) = _agg(x, src_p, dst_p, zeros)
    h = _dense(sums1, cnts, x, W1_l.T, b1_l.reshape(1, D), W1_r.T, relu=True)
    (sums2,) = _agg(h, src_p, dst_p, zeros)
    out = _dense(sums2, cnts, h, W2_l.T, b2_l.reshape(1, D), W2_r.T, relu=False)
    return out


# speedup vs baseline: 1.3700x; 1.0988x over previous
import functools

import jax
import jax.numpy as jnp
from jax import lax
from jax.experimental import pallas as pl
from jax.experimental.pallas import tpu as pltpu
from jax.experimental.pallas import tpu_sc as plsc

N = 10000
D = 128
E = 320000
NC = 2
NS = 16
NW = NC * NS
C = 128
J = 81
E_PAD = NW * J * C
N_SUB = 632
N_PAD = NS * N_SUB
CW = 16


def _agg_body(x_hbm, src_hbm, dst_hbm, z_hbm,
              sums_hbm,
              acc_sh, sring, dring, rows0, rows1, rows2,
              sg0, sg1, sg2, ssi0, ssi1, ssi2, si0, si1, si2):
    cid = lax.axis_index("c")
    sid = lax.axis_index("s")
    wid = sid * NC + cid
    r0 = sid * N_SUB
    e0 = wid * (J * C)
    pltpu.sync_copy(z_hbm, acc_sh.at[pl.ds(r0, N_SUB)])
    plsc.subcore_barrier()

    rows = (rows0, rows1, rows2)
    sg = (sg0, sg1, sg2)
    ssi = (ssi0, ssi1, ssi2)
    si = (si0, si1, si2)

    for b in range(3):
        pltpu.async_copy(src_hbm.at[pl.ds(e0 + b * C, C)], sring.at[b], ssi[b])
    for b in range(2):
        pltpu.async_copy(dst_hbm.at[pl.ds(e0 + b * C, C)], dring.at[b], si[b])
    for b in range(2):
        pltpu.make_async_copy(src_hbm.at[pl.ds(e0 + b * C, C)], sring.at[b],
                              ssi[b]).wait()
        pltpu.async_copy(x_hbm.at[sring.at[b]], rows[b], sg[b])

    def body(k, carry):
        for b in range(3):
            jj = 3 * k + b
            b2 = (b + 2) % 3
            pltpu.make_async_copy(x_hbm.at[sring.at[b]], rows[b], sg[b]).wait()

            @pl.when(jj + 2 < J)
            def _():
                pltpu.make_async_copy(
                    src_hbm.at[pl.ds(e0 + (jj + 2) * C, C)], sring.at[b2],
                    ssi[b2]).wait()
                pltpu.async_copy(x_hbm.at[sring.at[b2]], rows[b2], sg[b2])
                pltpu.async_copy(dst_hbm.at[pl.ds(e0 + (jj + 2) * C, C)],
                                 dring.at[b2], si[b2])

            @pl.when(jj + 3 < J)
            def _():
                pltpu.async_copy(src_hbm.at[pl.ds(e0 + (jj + 3) * C, C)],
                                 sring.at[b], ssi[b])

            pltpu.make_async_copy(dst_hbm.at[pl.ds(e0 + jj * C, C)],
                                  dring.at[b], si[b]).wait()
            pltpu.sync_copy(rows[b], acc_sh.at[dring.at[b]], add=True)
        return carry

    lax.fori_loop(0, J // 3, body, 0)
    plsc.subcore_barrier()
    pltpu.sync_copy(acc_sh.at[pl.ds(r0, N_SUB)], sums_hbm.at[cid, pl.ds(r0, N_SUB)])


def _count_body(dst_hbm, z_hbm, ones_hbm,
                cnt_hbm,
                cnt_sh, didx, ones_v, sem):
    cid = lax.axis_index("c")
    sid = lax.axis_index("s")
    wid = sid * NC + cid
    r0 = sid * N_SUB
    pltpu.sync_copy(z_hbm, cnt_sh.at[pl.ds(r0, N_SUB)])
    pltpu.sync_copy(dst_hbm.at[pl.ds(wid * (J * C), J * C)], didx)
    pltpu.sync_copy(ones_hbm, ones_v)
    plsc.subcore_barrier()

    def fire(j, carry):
        pltpu.async_copy(ones_v, cnt_sh.at[didx.at[pl.ds(j * C, C)]], sem,
                         add=True)
        return carry

    lax.fori_loop(0, J, fire, 0)

    def drain(j, carry):
        pltpu.make_async_copy(ones_v, cnt_sh.at[didx.at[pl.ds(0, C)]],
                              sem).wait()
        return carry

    lax.fori_loop(0, J, drain, 0)
    plsc.subcore_barrier()
    pltpu.sync_copy(cnt_sh.at[pl.ds(r0, N_SUB)], cnt_hbm.at[cid, pl.ds(r0, N_SUB)])


_MESH = plsc.VectorSubcoreMesh(core_axis_name="c", subcore_axis_name="s")

_agg = pl.kernel(
    _agg_body,
    mesh=_MESH,
    out_type=[jax.ShapeDtypeStruct((NC, N_PAD, D), jnp.float32)],
    scratch_types=[
        pltpu.VMEM_SHARED((N_PAD, D), jnp.float32),
        pltpu.VMEM((3, C), jnp.int32),
        pltpu.VMEM((3, C), jnp.int32),
        pltpu.VMEM((C, D), jnp.float32),
        pltpu.VMEM((C, D), jnp.float32),
        pltpu.VMEM((C, D), jnp.float32),
    ] + [pltpu.SemaphoreType.DMA] * 9,
)

_count = pl.kernel(
    _count_body,
    mesh=_MESH,
    out_type=[jax.ShapeDtypeStruct((NC, N_PAD, CW), jnp.float32)],
    scratch_types=[
        pltpu.VMEM_SHARED((N_PAD, CW), jnp.float32),
        pltpu.VMEM((J * C,), jnp.int32),
        pltpu.VMEM((C, CW), jnp.float32),
        pltpu.SemaphoreType.DMA,
    ],
    compiler_params=pltpu.CompilerParams(use_tc_tiling_on_sc=False),
)


def _dense_kernel(relu, s_ref, c_ref, x_ref, wl_ref, b_ref, wr_ref, o_ref):
    cnt = c_ref[0, :, 0:1] + c_ref[1, :, 0:1]
    mean = (s_ref[0] + s_ref[1]) / jnp.maximum(cnt, 1.0)
    r = (jnp.dot(mean, wl_ref[...], preferred_element_type=jnp.float32)
         + b_ref[...]
         + jnp.dot(x_ref[...], wr_ref[...], preferred_element_type=jnp.float32))
    if relu:
        r = jnp.maximum(r, 0.0)
    o_ref[...] = r


_BLK = 2000


def _dense(sums, cnts, x, wl_t, b, wr_t, relu):
    grid = (N // _BLK,)
    row_spec = pl.BlockSpec((_BLK, D), lambda i: (i, 0))
    sum_spec = pl.BlockSpec((NC, _BLK, D), lambda i: (0, i, 0))
    cnt_spec = pl.BlockSpec((NC, _BLK, CW), lambda i: (0, i, 0))
    w_spec = pl.BlockSpec((D, D), lambda i: (0, 0))
    b_spec = pl.BlockSpec((1, D), lambda i: (0, 0))
    return pl.pallas_call(
        functools.partial(_dense_kernel, relu),
        grid=grid,
        in_specs=[sum_spec, cnt_spec, row_spec, w_spec, b_spec, w_spec],
        out_specs=row_spec,
        out_shape=jax.ShapeDtypeStruct((N, D), jnp.float32),
    )(sums, cnts, x, wl_t, b, wr_t)


def kernel(x, edge_index, W1_l, b1_l, W1_r, W2_l, b2_l, W2_r):
    src = edge_index[0]
    dst = edge_index[1]
    npad = E_PAD - E
    pad_src = jnp.arange(npad, dtype=jnp.int32) & 8191
    pad_dst = N + (jnp.arange(npad, dtype=jnp.int32) % (N_PAD - N))
    src_p = jnp.concatenate([src, pad_src])
    dst_p = jnp.concatenate([dst, pad_dst])

    zeros = jnp.zeros((N_SUB, D), jnp.float32)
    zeros_c = jnp.zeros((N_SUB, CW), jnp.float32)
    ones = jnp.ones((C, CW), jnp.float32)

    (cnts,) = _count(dst_p, zeros_c, ones)
    (sums1,) = _agg(x, src_p, dst_p, zeros)
    h = _dense(sums1, cnts, x, W1_l.T, b1_l.reshape(1, D), W1_r.T, relu=True)
    (sums2,) = _agg(h, src_p, dst_p, zeros)
    out = _dense(sums2, cnts, h, W2_l.T, b2_l.reshape(1, D), W2_r.T, relu=False)
    return out
